# baseline (speedup 1.0000x reference)
import functools

import jax
import jax.numpy as jnp
from jax import lax
from jax.experimental import pallas as pl
from jax.experimental.pallas import tpu as pltpu
from jax.experimental.pallas import tpu_sc as plsc

N = 10000
E = 320000
D_IN = 128
D_HID = 256
D_OUT = 128
DF = 128

NC = 2
NS = 16
B = 128
E_PAD = 327680
NBD = E_PAD // (NC * NS) // B
N_PAD = 10240
PT = N_PAD // NS

_f32 = jnp.float32
_bf16 = jnp.bfloat16
_SC_PARAMS = pltpu.CompilerParams(use_tc_tiling_on_sc=False)


def _sc_mesh():
    return plsc.VectorSubcoreMesh(
        core_axis_name="c", subcore_axis_name="s", num_cores=NC, num_subcores=NS
    )


def _make_deg_kernel():
    @functools.partial(
        pl.kernel,
        out_type=jax.ShapeDtypeStruct((NC, N_PAD, 16), _f32),
        mesh=_sc_mesh(),
        compiler_params=_SC_PARAMS,
        scratch_types=[
            pltpu.VMEM((NBD, B), jnp.int32),
            pltpu.VMEM((B, 16), _f32),
            pltpu.VMEM_SHARED((N_PAD, 16), _f32),
        ],
    )
    def deg_kernel(dst4, ones_hbm, zeros_hbm, out, idxd, ones_v, acc):
        c = lax.axis_index("c")
        s = lax.axis_index("s")
        pltpu.sync_copy(dst4.at[c, s], idxd)
        pltpu.sync_copy(ones_hbm, ones_v)
        pltpu.sync_copy(zeros_hbm, acc.at[pl.ds(s * PT, PT)])
        plsc.subcore_barrier()

        def body(j, carry):
            pltpu.sync_copy(ones_v, acc.at[idxd.at[j]], add=True)
            return carry

        lax.fori_loop(0, NBD, body, 0)
        plsc.subcore_barrier()
        pltpu.sync_copy(acc.at[pl.ds(s * PT, PT)], out.at[c, pl.ds(s * PT, PT)])

    return deg_kernel


NBUF = 4


def _make_agg_kernel():
    @functools.partial(
        pl.kernel,
        out_type=jax.ShapeDtypeStruct((NC, N_PAD, DF), _bf16),
        mesh=_sc_mesh(),
        compiler_params=_SC_PARAMS,
        scratch_types=[
            pltpu.VMEM((NBD, B), jnp.int32),
            pltpu.VMEM((NBD, B), jnp.int32),
            [pltpu.VMEM((B, DF), _bf16) for _ in range(NBUF)],
            [pltpu.SemaphoreType.DMA for _ in range(NBUF)],
            [pltpu.SemaphoreType.DMA for _ in range(NBUF)],
            pltpu.VMEM_SHARED((N_PAD, DF), _bf16),
        ],
    )
    def agg_kernel(ytab, srcp, dstp, zrows, out, idxs, idxd, rv, gs, ss, acc):
        c = lax.axis_index("c")
        s = lax.axis_index("s")
        pltpu.sync_copy(srcp.at[c, s], idxs)
        pltpu.sync_copy(dstp.at[c, s], idxd)
        pltpu.sync_copy(zrows, acc.at[pl.ds(s * PT, PT)])
        plsc.subcore_barrier()

        for b in range(NBUF):
            pltpu.async_copy(ytab.at[idxs.at[b]], rv[b], gs[b])

        def body(t, carry):
            j0 = t * NBUF
            for b in range(NBUF):
                pltpu.make_async_copy(ytab.at[idxs.at[j0 + b]], rv[b], gs[b]).wait()
                pltpu.make_async_copy(rv[b], acc.at[idxd.at[j0 + b]], ss[b]).start(
                    add=True
                )
            for b in range(NBUF):

                @pl.when(j0 + b + NBUF < NBD)
                def _():
                    pltpu.make_async_copy(rv[b], acc.at[idxd.at[j0 + b]], ss[b]).wait()
                    pltpu.async_copy(ytab.at[idxs.at[j0 + b + NBUF]], rv[b], gs[b])

            return carry

        lax.fori_loop(0, NBD // NBUF, body, 0)
        for b in range(NBUF):
            pltpu.make_async_copy(rv[b], acc.at[idxd.at[NBD - NBUF + b]], ss[b]).wait()
        plsc.subcore_barrier()
        pltpu.sync_copy(acc.at[pl.ds(s * PT, PT)], out.at[c, pl.ds(s * PT, PT)])

    return agg_kernel


TR = 1000
NG = N // TR


def _dinv_block(degp_ref):
    deg = degp_ref[0, :, 0:1] + degp_ref[1, :, 0:1] + 1.0
    return lax.rsqrt(deg)


def _tc1_body(x_ref, degp_ref, out_ref):
    dinv = _dinv_block(degp_ref)
    u = (x_ref[...] * dinv).astype(_bf16)
    for c in range(NC):
        out_ref[c] = u


def _tc2_body(agg_ref, utab_ref, degp_ref, b1_ref, w1_ref, w2_ref, out_ref):
    dinv = _dinv_block(degp_ref)
    z = (
        agg_ref[0].astype(_f32)
        + agg_ref[1].astype(_f32)
        + utab_ref[0].astype(_f32)
    )
    w = dinv * z
    h = jnp.maximum(
        jnp.dot(w, w1_ref[...], preferred_element_type=_f32) + b1_ref[0, :], 0.0
    )
    y2 = (jnp.dot(h, w2_ref[...], preferred_element_type=_f32) * dinv).astype(_bf16)
    for c in range(NC):
        out_ref[c] = y2


def _tc3_body(agg_ref, ytab_ref, degp_ref, b2_ref, out_ref):
    dinv = _dinv_block(degp_ref)
    z = (
        agg_ref[0].astype(_f32)
        + agg_ref[1].astype(_f32)
        + ytab_ref[0].astype(_f32)
    )
    out_ref[...] = dinv * z + b2_ref[0, :]


def _tc1(x, degp):
    return pl.pallas_call(
        _tc1_body,
        grid=(NG,),
        in_specs=[
            pl.BlockSpec((TR, D_IN), lambda i: (i, 0)),
            pl.BlockSpec((NC, TR, 16), lambda i: (0, i, 0)),
        ],
        out_specs=pl.BlockSpec((NC, TR, DF), lambda i: (0, i, 0)),
        out_shape=jax.ShapeDtypeStruct((NC, N_PAD, DF), _bf16),
    )(x, degp)


def _tc2(agg1, utab, degp, b1, W1, W2):
    return pl.pallas_call(
        _tc2_body,
        grid=(NG,),
        in_specs=[
            pl.BlockSpec((NC, TR, DF), lambda i: (0, i, 0)),
            pl.BlockSpec((NC, TR, DF), lambda i: (0, i, 0)),
            pl.BlockSpec((NC, TR, 16), lambda i: (0, i, 0)),
            pl.BlockSpec((1, D_HID), lambda i: (0, 0)),
            pl.BlockSpec((D_IN, D_HID), lambda i: (0, 0)),
            pl.BlockSpec((D_HID, D_OUT), lambda i: (0, 0)),
        ],
        out_specs=pl.BlockSpec((NC, TR, DF), lambda i: (0, i, 0)),
        out_shape=jax.ShapeDtypeStruct((NC, N_PAD, DF), _bf16),
    )(agg1, utab, degp, b1, W1, W2)


def _tc3(agg2, ytab2, degp, b2):
    return pl.pallas_call(
        _tc3_body,
        grid=(NG,),
        in_specs=[
            pl.BlockSpec((NC, TR, DF), lambda i: (0, i, 0)),
            pl.BlockSpec((NC, TR, DF), lambda i: (0, i, 0)),
            pl.BlockSpec((NC, TR, 16), lambda i: (0, i, 0)),
            pl.BlockSpec((1, D_OUT), lambda i: (0, 0)),
        ],
        out_specs=pl.BlockSpec((TR, D_OUT), lambda i: (i, 0)),
        out_shape=jax.ShapeDtypeStruct((N, D_OUT), _f32),
    )(agg2, ytab2, degp, b2)


def kernel(x, edge_index, W1, b1, W2, b2):
    src = edge_index[0].astype(jnp.int32)
    dst = edge_index[1].astype(jnp.int32)

    pad = E_PAD - E
    srcp = jnp.concatenate([src, jnp.zeros((pad,), jnp.int32)]).reshape(NC, NS, NBD, B)
    srcp = srcp + (jnp.arange(NC, dtype=jnp.int32) * N_PAD)[:, None, None, None]
    dstp = jnp.concatenate([dst, jnp.full((pad,), N, jnp.int32)]).reshape(NC, NS, NBD, B)

    ones16 = jnp.ones((B, 16), _f32)
    zeros16 = jnp.zeros((PT, 16), _f32)
    zrows = jnp.zeros((PT, DF), _bf16)

    degp = _make_deg_kernel()(dstp, ones16, zeros16)
    utab = _tc1(x, degp)
    agg1 = _make_agg_kernel()(utab.reshape(NC * N_PAD, DF), srcp, dstp, zrows)
    ytab2 = _tc2(agg1, utab, degp, b1.reshape(1, D_HID), W1, W2)
    agg2 = _make_agg_kernel()(ytab2.reshape(NC * N_PAD, DF), srcp, dstp, zrows)
    return _tc3(agg2, ytab2, degp, b2.reshape(1, D_OUT))


# speedup vs baseline: 1.1560x; 1.1560x over previous
import functools

import jax
import jax.numpy as jnp
from jax import lax
from jax.experimental import pallas as pl
from jax.experimental.pallas import tpu as pltpu
from jax.experimental.pallas import tpu_sc as plsc

N = 10000
E = 320000
D_IN = 128
D_HID = 256
D_OUT = 128
DF = 128

NC = 2
NS = 16
B = 128
E_PAD = 327680
NBD = E_PAD // (NC * NS) // B
N_PAD = 10240
PT = N_PAD // NS

_f32 = jnp.float32
_bf16 = jnp.bfloat16
_SC_PARAMS = pltpu.CompilerParams(use_tc_tiling_on_sc=False)


def _sc_mesh():
    return plsc.VectorSubcoreMesh(
        core_axis_name="c", subcore_axis_name="s", num_cores=NC, num_subcores=NS
    )


def _make_deg_kernel():
    @functools.partial(
        pl.kernel,
        out_type=jax.ShapeDtypeStruct((NC, N_PAD, 16), _f32),
        mesh=_sc_mesh(),
        compiler_params=_SC_PARAMS,
        scratch_types=[
            pltpu.VMEM((NBD, B), jnp.int32),
            pltpu.VMEM((B, 16), _f32),
            pltpu.VMEM_SHARED((N_PAD, 16), _f32),
        ],
    )
    def deg_kernel(dst4, ones_hbm, zeros_hbm, out, idxd, ones_v, acc):
        c = lax.axis_index("c")
        s = lax.axis_index("s")
        pltpu.sync_copy(dst4.at[c, s], idxd)
        pltpu.sync_copy(ones_hbm, ones_v)
        pltpu.sync_copy(zeros_hbm, acc.at[pl.ds(s * PT, PT)])
        plsc.subcore_barrier()

        def body(j, carry):
            pltpu.sync_copy(ones_v, acc.at[idxd.at[j]], add=True)
            return carry

        lax.fori_loop(0, NBD, body, 0)
        plsc.subcore_barrier()
        pltpu.sync_copy(acc.at[pl.ds(s * PT, PT)], out.at[c, pl.ds(s * PT, PT)])

    return deg_kernel


NBUF = 4


def _make_agg_kernel():
    @functools.partial(
        pl.kernel,
        out_type=jax.ShapeDtypeStruct((NC, N_PAD, DF), _bf16),
        mesh=_sc_mesh(),
        compiler_params=_SC_PARAMS,
        scratch_types=[
            pltpu.VMEM((NBD, B), jnp.int32),
            pltpu.VMEM((NBD, B), jnp.int32),
            [pltpu.VMEM((B, DF), _bf16) for _ in range(NBUF)],
            [pltpu.SemaphoreType.DMA for _ in range(NBUF)],
            [pltpu.SemaphoreType.DMA for _ in range(NBUF)],
            pltpu.VMEM_SHARED((N_PAD, DF), _bf16),
        ],
    )
    def agg_kernel(ytab, srcp, dstp, zrows, out, idxs, idxd, rv, gs, ss, acc):
        c = lax.axis_index("c")
        s = lax.axis_index("s")
        pltpu.sync_copy(srcp.at[c, s], idxs)
        pltpu.sync_copy(dstp.at[c, s], idxd)
        pltpu.sync_copy(zrows, acc.at[pl.ds(s * PT, PT)])
        plsc.subcore_barrier()

        for b in range(NBUF):
            pltpu.async_copy(ytab.at[idxs.at[b]], rv[b], gs[b])

        def body(t, carry):
            j0 = t * NBUF
            for b in range(NBUF):
                pltpu.make_async_copy(ytab.at[idxs.at[j0 + b]], rv[b], gs[b]).wait()
                pltpu.make_async_copy(rv[b], acc.at[idxd.at[j0 + b]], ss[b]).start(
                    add=True
                )
            for b in range(NBUF):

                @pl.when(j0 + b + NBUF < NBD)
                def _():
                    pltpu.make_async_copy(rv[b], acc.at[idxd.at[j0 + b]], ss[b]).wait()
                    pltpu.async_copy(ytab.at[idxs.at[j0 + b + NBUF]], rv[b], gs[b])

            return carry

        lax.fori_loop(0, NBD // NBUF, body, 0)
        for b in range(NBUF):
            pltpu.make_async_copy(rv[b], acc.at[idxd.at[NBD - NBUF + b]], ss[b]).wait()
        plsc.subcore_barrier()
        pltpu.sync_copy(acc.at[pl.ds(s * PT, PT)], out.at[c, pl.ds(s * PT, PT)])

    return agg_kernel


TR = 1000
NG = N // TR


def _dinv_block(degp_ref):
    deg = degp_ref[0, :, 0:1] + degp_ref[1, :, 0:1] + 1.0
    return lax.rsqrt(deg)


def _tc1_body(x_ref, degp_ref, out_ref):
    dinv = _dinv_block(degp_ref)
    out_ref[...] = (x_ref[...] * dinv).astype(_bf16)


def _tc2_body(agg_ref, utab_ref, degp_ref, b1_ref, w1_ref, w2_ref, out_ref):
    dinv = _dinv_block(degp_ref)
    z = (
        agg_ref[0].astype(_f32)
        + agg_ref[1].astype(_f32)
        + utab_ref[...].astype(_f32)
    )
    w = dinv * z
    h = jnp.maximum(
        jnp.dot(w, w1_ref[...], preferred_element_type=_f32) + b1_ref[0, :], 0.0
    )
    y2 = jnp.dot(h, w2_ref[...], preferred_element_type=_f32) * dinv
    out_ref[...] = y2.astype(_bf16)


def _tc3_body(agg_ref, ytab_ref, degp_ref, b2_ref, out_ref):
    dinv = _dinv_block(degp_ref)
    z = (
        agg_ref[0].astype(_f32)
        + agg_ref[1].astype(_f32)
        + ytab_ref[...].astype(_f32)
    )
    out_ref[...] = dinv * z + b2_ref[0, :]


def _tc1(x, degp):
    return pl.pallas_call(
        _tc1_body,
        grid=(NG,),
        in_specs=[
            pl.BlockSpec((TR, D_IN), lambda i: (i, 0)),
            pl.BlockSpec((NC, TR, 16), lambda i: (0, i, 0)),
        ],
        out_specs=pl.BlockSpec((TR, DF), lambda i: (i, 0)),
        out_shape=jax.ShapeDtypeStruct((N_PAD, DF), _bf16),
    )(x, degp)


def _tc2(agg1, utab, degp, b1, W1, W2):
    return pl.pallas_call(
        _tc2_body,
        grid=(NG,),
        in_specs=[
            pl.BlockSpec((NC, TR, DF), lambda i: (0, i, 0)),
            pl.BlockSpec((TR, DF), lambda i: (i, 0)),
            pl.BlockSpec((NC, TR, 16), lambda i: (0, i, 0)),
            pl.BlockSpec((1, D_HID), lambda i: (0, 0)),
            pl.BlockSpec((D_IN, D_HID), lambda i: (0, 0)),
            pl.BlockSpec((D_HID, D_OUT), lambda i: (0, 0)),
        ],
        out_specs=pl.BlockSpec((TR, DF), lambda i: (i, 0)),
        out_shape=jax.ShapeDtypeStruct((N_PAD, DF), _bf16),
    )(agg1, utab, degp, b1, W1, W2)


def _tc3(agg2, ytab2, degp, b2):
    return pl.pallas_call(
        _tc3_body,
        grid=(NG,),
        in_specs=[
            pl.BlockSpec((NC, TR, DF), lambda i: (0, i, 0)),
            pl.BlockSpec((TR, DF), lambda i: (i, 0)),
            pl.BlockSpec((NC, TR, 16), lambda i: (0, i, 0)),
            pl.BlockSpec((1, D_OUT), lambda i: (0, 0)),
        ],
        out_specs=pl.BlockSpec((TR, D_OUT), lambda i: (i, 0)),
        out_shape=jax.ShapeDtypeStruct((N, D_OUT), _f32),
    )(agg2, ytab2, degp, b2)


def kernel(x, edge_index, W1, b1, W2, b2):
    src = edge_index[0].astype(jnp.int32)
    dst = edge_index[1].astype(jnp.int32)

    pad = E_PAD - E
    srcp = jnp.concatenate([src, jnp.zeros((pad,), jnp.int32)]).reshape(NC, NS, NBD, B)
    pad_dst = N + jnp.arange(pad, dtype=jnp.int32) % (N_PAD - N)
    dstp = jnp.concatenate([dst, pad_dst]).reshape(NC, NS, NBD, B)

    ones16 = jnp.ones((B, 16), _f32)
    zeros16 = jnp.zeros((PT, 16), _f32)
    zrows = jnp.zeros((PT, DF), _bf16)

    degp = _make_deg_kernel()(dstp, ones16, zeros16)
    utab = _tc1(x, degp)
    agg1 = _make_agg_kernel()(utab, srcp, dstp, zrows)
    ytab2 = _tc2(agg1, utab, degp, b1.reshape(1, D_HID), W1, W2)
    agg2 = _make_agg_kernel()(ytab2, srcp, dstp, zrows)
    return _tc3(agg2, ytab2, degp, b2.reshape(1, D_OUT))


# speedup vs baseline: 1.4262x; 1.2338x over previous
import functools

import jax
import jax.numpy as jnp
from jax import lax
from jax.experimental import pallas as pl
from jax.experimental.pallas import tpu as pltpu
from jax.experimental.pallas import tpu_sc as plsc

N = 10000
E = 320000
D_IN = 128
D_HID = 256
D_OUT = 128

NC = 2
NS = 16
B = 128
E_PAD = 327680
NB = E_PAD // NS // B
NBD = E_PAD // (NC * NS) // B
N_PAD = 10240
PT = N_PAD // NS
DQ = 64

_f32 = jnp.float32
_bf16 = jnp.bfloat16
_SC_PARAMS = pltpu.CompilerParams(use_tc_tiling_on_sc=False)


def _sc_mesh():
    return plsc.VectorSubcoreMesh(
        core_axis_name="c", subcore_axis_name="s", num_cores=NC, num_subcores=NS
    )


def _make_deg_kernel():
    @functools.partial(
        pl.kernel,
        out_type=jax.ShapeDtypeStruct((NC, N_PAD, 16), _f32),
        mesh=_sc_mesh(),
        compiler_params=_SC_PARAMS,
        scratch_types=[
            pltpu.VMEM((NBD, B), jnp.int32),
            pltpu.VMEM((B, 16), _f32),
            pltpu.VMEM_SHARED((N_PAD, 16), _f32),
        ],
    )
    def deg_kernel(dst4, ones_hbm, zeros_hbm, out, idxd, ones_v, acc):
        c = lax.axis_index("c")
        s = lax.axis_index("s")
        pltpu.sync_copy(dst4.at[c, s], idxd)
        pltpu.sync_copy(ones_hbm, ones_v)
        pltpu.sync_copy(zeros_hbm, acc.at[pl.ds(s * PT, PT)])
        plsc.subcore_barrier()

        def body(j, carry):
            pltpu.sync_copy(ones_v, acc.at[idxd.at[j]], add=True)
            return carry

        lax.fori_loop(0, NBD, body, 0)
        plsc.subcore_barrier()
        pltpu.sync_copy(acc.at[pl.ds(s * PT, PT)], out.at[c, pl.ds(s * PT, PT)])

    return deg_kernel


NBUF = 4


def _make_agg_kernel(NP):
    @functools.partial(
        pl.kernel,
        out_type=jax.ShapeDtypeStruct((NC * NP, N_PAD, DQ), _bf16),
        mesh=_sc_mesh(),
        compiler_params=_SC_PARAMS,
        scratch_types=[
            pltpu.VMEM((NB, B), jnp.int32),
            pltpu.VMEM((NB, B), jnp.int32),
            [pltpu.VMEM((B, DQ), _bf16) for _ in range(NBUF)],
            [pltpu.SemaphoreType.DMA for _ in range(NBUF)],
            [pltpu.SemaphoreType.DMA for _ in range(NBUF)],
            pltpu.VMEM_SHARED((N_PAD, DQ), _bf16),
        ],
    )
    def agg_kernel(ytab, srcp, dstp, zrows, out, idxs, idxd, rv, gs, ss, acc):
        c = lax.axis_index("c")
        s = lax.axis_index("s")
        pltpu.sync_copy(dstp.at[s], idxd)
        for p in range(NP):
            q = c * NP + p
            pltpu.sync_copy(srcp.at[q, s], idxs)
            pltpu.sync_copy(zrows, acc.at[pl.ds(s * PT, PT)])
            plsc.subcore_barrier()

            for b in range(NBUF):
                pltpu.async_copy(ytab.at[idxs.at[b]], rv[b], gs[b])

            def body(t, carry):
                j0 = t * NBUF
                for b in range(NBUF):
                    pltpu.make_async_copy(ytab.at[idxs.at[j0 + b]], rv[b], gs[b]).wait()
                    pltpu.make_async_copy(rv[b], acc.at[idxd.at[j0 + b]], ss[b]).start(
                        add=True
                    )
                for b in range(NBUF):

                    @pl.when(j0 + b + NBUF < NB)
                    def _():
                        pltpu.make_async_copy(
                            rv[b], acc.at[idxd.at[j0 + b]], ss[b]
                        ).wait()
                        pltpu.async_copy(ytab.at[idxs.at[j0 + b + NBUF]], rv[b], gs[b])

                return carry

            lax.fori_loop(0, NB // NBUF, body, 0)
            for b in range(NBUF):
                pltpu.make_async_copy(rv[b], acc.at[idxd.at[NB - NBUF + b]], ss[b]).wait()
            plsc.subcore_barrier()
            pltpu.sync_copy(acc.at[pl.ds(s * PT, PT)], out.at[q, pl.ds(s * PT, PT)])
            plsc.subcore_barrier()

    return agg_kernel


TR = 1000
NG = N // TR
NQ = D_IN // DQ


def _dinv_block(degp_ref):
    deg = degp_ref[0, :, 0:1] + degp_ref[1, :, 0:1] + 1.0
    return lax.rsqrt(deg)


def _tc1_body(x_ref, degp_ref, out_ref):
    dinv = _dinv_block(degp_ref)
    u = (x_ref[...] * dinv).astype(_bf16)
    for q in range(NQ):
        out_ref[q] = u[:, q * DQ:(q + 1) * DQ]


def _tc2_body(agg_ref, utab_ref, degp_ref, b1_ref, w1_ref, w2_ref, out_ref):
    dinv = _dinv_block(degp_ref)
    wq = [
        dinv * (agg_ref[q].astype(_f32) + utab_ref[q].astype(_f32)) for q in range(NQ)
    ]
    w = jnp.concatenate(wq, axis=1)
    h = jnp.maximum(
        jnp.dot(w, w1_ref[...], preferred_element_type=_f32) + b1_ref[0, :], 0.0
    )
    y2 = ((jnp.dot(h, w2_ref[...], preferred_element_type=_f32)) * dinv).astype(_bf16)
    for q in range(NQ):
        out_ref[q] = y2[:, q * DQ:(q + 1) * DQ]


def _tc3_body(agg_ref, ytab_ref, degp_ref, b2_ref, out_ref):
    dinv = _dinv_block(degp_ref)
    oq = [
        dinv * (agg_ref[q].astype(_f32) + ytab_ref[q].astype(_f32))
        + b2_ref[0, q * DQ:(q + 1) * DQ]
        for q in range(NQ)
    ]
    out_ref[...] = jnp.concatenate(oq, axis=1)


def _tc1(x, degp):
    return pl.pallas_call(
        _tc1_body,
        grid=(NG,),
        in_specs=[
            pl.BlockSpec((TR, D_IN), lambda i: (i, 0)),
            pl.BlockSpec((NC, TR, 16), lambda i: (0, i, 0)),
        ],
        out_specs=pl.BlockSpec((NQ, TR, DQ), lambda i: (0, i, 0)),
        out_shape=jax.ShapeDtypeStruct((NQ, N, DQ), _bf16),
    )(x, degp)


def _tc2(agg1, utab, degp, b1, W1, W2):
    return pl.pallas_call(
        _tc2_body,
        grid=(NG,),
        in_specs=[
            pl.BlockSpec((NQ, TR, DQ), lambda i: (0, i, 0)),
            pl.BlockSpec((NQ, TR, DQ), lambda i: (0, i, 0)),
            pl.BlockSpec((NC, TR, 16), lambda i: (0, i, 0)),
            pl.BlockSpec((1, D_HID), lambda i: (0, 0)),
            pl.BlockSpec((D_IN, D_HID), lambda i: (0, 0)),
            pl.BlockSpec((D_HID, D_OUT), lambda i: (0, 0)),
        ],
        out_specs=pl.BlockSpec((NQ, TR, DQ), lambda i: (0, i, 0)),
        out_shape=jax.ShapeDtypeStruct((NQ, N, DQ), _bf16),
    )(agg1, utab, degp, b1, W1, W2)


def _tc3(agg2, ytab2, degp, b2):
    return pl.pallas_call(
        _tc3_body,
        grid=(NG,),
        in_specs=[
            pl.BlockSpec((NQ, TR, DQ), lambda i: (0, i, 0)),
            pl.BlockSpec((NQ, TR, DQ), lambda i: (0, i, 0)),
            pl.BlockSpec((NC, TR, 16), lambda i: (0, i, 0)),
            pl.BlockSpec((1, D_OUT), lambda i: (0, 0)),
        ],
        out_specs=pl.BlockSpec((TR, D_OUT), lambda i: (i, 0)),
        out_shape=jax.ShapeDtypeStruct((N, D_OUT), _f32),
    )(agg2, ytab2, degp, b2)


def kernel(x, edge_index, W1, b1, W2, b2):
    src = edge_index[0].astype(jnp.int32)
    dst = edge_index[1].astype(jnp.int32)

    pad = E_PAD - E
    srcz = jnp.concatenate([src, jnp.zeros((pad,), jnp.int32)])
    dstz = jnp.concatenate([dst, N + jnp.arange(pad, dtype=jnp.int32) % (N_PAD - N)])
    srcp = jnp.stack([srcz + q * N for q in range(NQ)]).reshape(NQ, NS, NB, B)
    dstp = dstz.reshape(NS, NB, B)
    dst4 = dstz.reshape(NC, NS, NBD, B)

    ones16 = jnp.ones((B, 16), _f32)
    zeros16 = jnp.zeros((PT, 16), _f32)
    zrows = jnp.zeros((PT, DQ), _bf16)

    degp = _make_deg_kernel()(dst4, ones16, zeros16)
    utab = _tc1(x, degp)
    agg1 = _make_agg_kernel(NQ // NC)(utab.reshape(NQ * N, DQ), srcp, dstp, zrows)
    ytab2 = _tc2(agg1, utab, degp, b1.reshape(1, D_HID), W1, W2)
    agg2 = _make_agg_kernel(NQ // NC)(ytab2.reshape(NQ * N, DQ), srcp, dstp, zrows)
    return _tc3(agg2, ytab2, degp, b2.reshape(1, D_OUT))
